# baseline (speedup 1.0000x reference)
import functools
import math

import jax
import jax.numpy as jnp
from jax import lax
from jax.experimental import pallas as pl
from jax.experimental.pallas import tpu as pltpu
from jax.experimental.pallas import tpu_sc as plsc

D_MODEL = 64
SCALE = math.sqrt(D_MODEL)
NUM_CORES = 2
NUM_SUBCORES = 16
NUM_WORKERS = NUM_CORES * NUM_SUBCORES
LANES = 16
VOCAB = 1000000
VBLK = 128
N_VBLK_SLOTS = 7840
VBLK_PER_W = N_VBLK_SLOTS // NUM_WORKERS
V0_MAX = (VOCAB // VBLK) * VBLK
WIDE_ROWS = V0_MAX + VBLK


def _mesh():
    return plsc.VectorSubcoreMesh(core_axis_name="c", subcore_axis_name="s")


def _widen(table_t):
    @functools.partial(
        pl.kernel,
        mesh=_mesh(),
        out_type=jax.ShapeDtypeStruct((WIDE_ROWS, 128), jnp.float32),
        scratch_types=[
            pltpu.VMEM((D_MODEL, VBLK), jnp.float32),
            pltpu.VMEM((D_MODEL, VBLK), jnp.float32),
            pltpu.VMEM((VBLK, 128), jnp.float32),
            pltpu.VMEM((VBLK, 128), jnp.float32),
            pltpu.SemaphoreType.DMA,
            pltpu.SemaphoreType.DMA,
            pltpu.SemaphoreType.DMA,
            pltpu.SemaphoreType.DMA,
        ],
        compiler_params=pltpu.CompilerParams(needs_layout_passes=False),
    )
    def k1(tt_hbm, wide_hbm, src0, src1, dst0, dst1, g0, g1, w0, w1):
        wid = lax.axis_index("s") * NUM_CORES + lax.axis_index("c")
        srcs = (src0, src1)
        dsts = (dst0, dst1)
        gsems = (g0, g1)
        wsems = (w0, w1)
        iotas = [lax.iota(jnp.int32, LANES) + j * LANES
                 for j in range(VBLK // LANES)]

        def v0_of(i):
            return jnp.minimum((i * NUM_WORKERS + wid) * VBLK, V0_MAX)

        def start_load(i, b):
            pltpu.async_copy(
                tt_hbm.at[:, pl.ds(v0_of(i), VBLK)], srcs[b], gsems[b])

        for b in range(2):
            start_load(b, b)

        def blk_body(r, carry):
            for b in range(2):
                i = r * 2 + b
                pltpu.make_async_copy(
                    tt_hbm.at[:, pl.ds(0, VBLK)], srcs[b], gsems[b]).wait()

                @pl.when(i >= 2)
                def _():
                    pltpu.make_async_copy(
                        dsts[b], wide_hbm.at[pl.ds(0, VBLK), :],
                        wsems[b]).wait()

                def row_body(d, c):
                    d_idx = jnp.full((LANES,), d, dtype=jnp.int32)
                    for j in range(VBLK // LANES):
                        v = srcs[b][d, pl.ds(j * LANES, LANES)]
                        plsc.store_scatter(dsts[b], [iotas[j], d_idx], v)
                    return c

                lax.fori_loop(0, D_MODEL, row_body, 0)

                @pl.when(i + 2 < VBLK_PER_W)
                def _():
                    start_load(i + 2, b)

                pltpu.async_copy(
                    dsts[b], wide_hbm.at[pl.ds(v0_of(i), VBLK), :], wsems[b])
            return carry

        lax.fori_loop(0, VBLK_PER_W // 2, blk_body, 0)
        i_last = VBLK_PER_W - 1
        b = i_last % 2
        pltpu.make_async_copy(
            tt_hbm.at[:, pl.ds(0, VBLK)], srcs[b], gsems[b]).wait()
        pltpu.make_async_copy(
            dsts[b], wide_hbm.at[pl.ds(0, VBLK), :], wsems[b]).wait()

        def row_body_last(d, c):
            d_idx = jnp.full((LANES,), d, dtype=jnp.int32)
            for j in range(VBLK // LANES):
                v = srcs[b][d, pl.ds(j * LANES, LANES)]
                plsc.store_scatter(dsts[b], [iotas[j], d_idx], v)
            return c

        lax.fori_loop(0, D_MODEL, row_body_last, 0)
        pltpu.async_copy(
            dsts[b], wide_hbm.at[pl.ds(v0_of(i_last), VBLK), :], wsems[b])

        for bb in range(2):
            pltpu.make_async_copy(
                dsts[bb], wide_hbm.at[pl.ds(0, VBLK), :], wsems[bb]).wait()

    return k1(table_t)


def _gather(x_t, wide):
    n_pos, n_batch = x_t.shape
    rblk = n_batch // NUM_WORKERS

    @functools.partial(
        pl.kernel,
        mesh=_mesh(),
        out_type=jax.ShapeDtypeStruct((n_pos, D_MODEL, n_batch), jnp.float32),
        scratch_types=[
            pltpu.VMEM((n_pos, rblk), jnp.int32),
            pltpu.VMEM((rblk, 128), jnp.float32),
            pltpu.VMEM((rblk, 128), jnp.float32),
            pltpu.VMEM((D_MODEL, rblk), jnp.float32),
            pltpu.VMEM((D_MODEL, rblk), jnp.float32),
            pltpu.SemaphoreType.DMA,
            pltpu.SemaphoreType.DMA,
            pltpu.SemaphoreType.DMA,
            pltpu.SemaphoreType.DMA,
        ],
        compiler_params=pltpu.CompilerParams(needs_layout_passes=False),
    )
    def k2(xt_hbm, wide_hbm, out_hbm, idxv, in0, in1, ob0, ob1,
           g0, g1, w0, w1):
        wid = lax.axis_index("s") * NUM_CORES + lax.axis_index("c")
        r0 = wid * rblk
        ins = (in0, in1)
        obs = (ob0, ob1)
        gsems = (g0, g1)
        wsems = (w0, w1)
        iotas = [lax.iota(jnp.int32, LANES) + j * LANES
                 for j in range(D_MODEL // LANES)]

        pltpu.sync_copy(xt_hbm.at[:, pl.ds(r0, rblk)], idxv)

        def start_gather(t, b):
            pltpu.async_copy(wide_hbm.at[idxv.at[t]], ins[b], gsems[b])

        for b in range(2):
            start_gather(b, b)

        def t_body(r, carry):
            for b in range(2):
                t = r * 2 + b
                pltpu.make_async_copy(
                    wide_hbm.at[idxv.at[0]], ins[b], gsems[b]).wait()

                @pl.when(t >= 2)
                def _():
                    pltpu.make_async_copy(
                        obs[b], out_hbm.at[0, :, pl.ds(0, rblk)],
                        wsems[b]).wait()

                def row_body(rr, c):
                    r_idx = jnp.full((LANES,), rr, dtype=jnp.int32)
                    for j in range(D_MODEL // LANES):
                        v = ins[b][rr, pl.ds(j * LANES, LANES)] * SCALE
                        plsc.store_scatter(obs[b], [iotas[j], r_idx], v)
                    return c

                lax.fori_loop(0, rblk, row_body, 0)

                @pl.when(t + 2 < n_pos)
                def _():
                    start_gather(t + 2, b)

                pltpu.async_copy(
                    obs[b], out_hbm.at[t, :, pl.ds(r0, rblk)], wsems[b])
            return carry

        lax.fori_loop(0, n_pos // 2, t_body, 0)

        for bb in range(2):
            pltpu.make_async_copy(
                obs[bb], out_hbm.at[0, :, pl.ds(0, rblk)], wsems[bb]).wait()

    return k2(x_t, wide)


def kernel(x, table):
    x_t = x.astype(jnp.int32).T
    table_t = table.T
    wide = _widen(table_t)
    out_t = _gather(x_t, wide)
    return jnp.transpose(out_t, (2, 0, 1))


# speedup vs baseline: 2.3324x; 2.3324x over previous
import functools
import math

import jax
import jax.numpy as jnp
from jax import lax
from jax.experimental import pallas as pl
from jax.experimental.pallas import tpu as pltpu
from jax.experimental.pallas import tpu_sc as plsc

D_MODEL = 64
SCALE = math.sqrt(D_MODEL)
NUM_CORES = 2
NUM_SUBCORES = 16
NUM_WORKERS = NUM_CORES * NUM_SUBCORES
LANES = 16
VOCAB = 1000000
VBLK = 128
N_VBLK_SLOTS = 7840
VBLK_PER_W = N_VBLK_SLOTS // NUM_WORKERS
V0_MAX = (VOCAB // VBLK) * VBLK
WIDE_ROWS = V0_MAX + VBLK


def _mesh():
    return plsc.VectorSubcoreMesh(core_axis_name="c", subcore_axis_name="s")


def _widen(table_t):
    @functools.partial(
        pl.kernel,
        mesh=_mesh(),
        out_type=jax.ShapeDtypeStruct((WIDE_ROWS, 128), jnp.float32),
        scratch_types=[
            pltpu.VMEM((D_MODEL, VBLK), jnp.float32),
            pltpu.VMEM((D_MODEL, VBLK), jnp.float32),
            pltpu.VMEM((VBLK, 128), jnp.float32),
            pltpu.VMEM((VBLK, 128), jnp.float32),
            pltpu.SemaphoreType.DMA,
            pltpu.SemaphoreType.DMA,
            pltpu.SemaphoreType.DMA,
            pltpu.SemaphoreType.DMA,
        ],
        compiler_params=pltpu.CompilerParams(needs_layout_passes=False),
    )
    def k1(tt_hbm, wide_hbm, src0, src1, dst0, dst1, g0, g1, w0, w1):
        wid = lax.axis_index("s") * NUM_CORES + lax.axis_index("c")
        srcs = (src0, src1)
        dsts = (dst0, dst1)
        gsems = (g0, g1)
        wsems = (w0, w1)
        iota = lax.iota(jnp.int32, LANES)
        rbs = [iota + di * LANES for di in range(D_MODEL // LANES)]

        def v0_of(i):
            return jnp.minimum((i * NUM_WORKERS + wid) * VBLK, V0_MAX)

        def start_load(i, b):
            pltpu.async_copy(
                tt_hbm.at[:, pl.ds(v0_of(i), VBLK)], srcs[b], gsems[b])

        for b in range(2):
            start_load(b, b)

        def blk_body(r, carry):
            for b in range(2):
                i = r * 2 + b
                pltpu.make_async_copy(
                    tt_hbm.at[:, pl.ds(0, VBLK)], srcs[b], gsems[b]).wait()

                @pl.when(i >= 2)
                def _():
                    pltpu.make_async_copy(
                        dsts[b], wide_hbm.at[pl.ds(0, VBLK), :],
                        wsems[b]).wait()

                def diag_body(c, cr):
                    pc = (iota + c) & (LANES - 1)
                    for di in range(D_MODEL // LANES):
                        rb = rbs[di]
                        for vj in range(VBLK // LANES):
                            col = pc + vj * LANES
                            v = plsc.load_gather(srcs[b], [rb, col])
                            plsc.store_scatter(dsts[b], [col, rb], v)
                    return cr

                lax.fori_loop(0, LANES, diag_body, 0)

                @pl.when(i + 2 < VBLK_PER_W)
                def _():
                    start_load(i + 2, b)

                pltpu.async_copy(
                    dsts[b], wide_hbm.at[pl.ds(v0_of(i), VBLK), :], wsems[b])
            return carry

        lax.fori_loop(0, VBLK_PER_W // 2, blk_body, 0)
        i_last = VBLK_PER_W - 1
        b = i_last % 2
        pltpu.make_async_copy(
            tt_hbm.at[:, pl.ds(0, VBLK)], srcs[b], gsems[b]).wait()
        pltpu.make_async_copy(
            dsts[b], wide_hbm.at[pl.ds(0, VBLK), :], wsems[b]).wait()

        def diag_body_last(c, cr):
            pc = (iota + c) & (LANES - 1)
            for di in range(D_MODEL // LANES):
                rb = rbs[di]
                for vj in range(VBLK // LANES):
                    col = pc + vj * LANES
                    v = plsc.load_gather(srcs[b], [rb, col])
                    plsc.store_scatter(dsts[b], [col, rb], v)
            return cr

        lax.fori_loop(0, LANES, diag_body_last, 0)
        pltpu.async_copy(
            dsts[b], wide_hbm.at[pl.ds(v0_of(i_last), VBLK), :], wsems[b])

        for bb in range(2):
            pltpu.make_async_copy(
                dsts[bb], wide_hbm.at[pl.ds(0, VBLK), :], wsems[bb]).wait()

    return k1(table_t)


def _gather(x_t, wide):
    n_pos, n_batch = x_t.shape
    rblk = n_batch // NUM_WORKERS

    @functools.partial(
        pl.kernel,
        mesh=_mesh(),
        out_type=jax.ShapeDtypeStruct((n_pos, D_MODEL, n_batch), jnp.float32),
        scratch_types=[
            pltpu.VMEM((n_pos, rblk), jnp.int32),
            pltpu.VMEM((rblk, 128), jnp.float32),
            pltpu.VMEM((rblk, 128), jnp.float32),
            pltpu.VMEM((D_MODEL, rblk), jnp.float32),
            pltpu.VMEM((D_MODEL, rblk), jnp.float32),
            pltpu.SemaphoreType.DMA,
            pltpu.SemaphoreType.DMA,
            pltpu.SemaphoreType.DMA,
            pltpu.SemaphoreType.DMA,
        ],
        compiler_params=pltpu.CompilerParams(needs_layout_passes=False),
    )
    def k2(xt_hbm, wide_hbm, out_hbm, idxv, in0, in1, ob0, ob1,
           g0, g1, w0, w1):
        wid = lax.axis_index("s") * NUM_CORES + lax.axis_index("c")
        r0 = wid * rblk
        ins = (in0, in1)
        obs = (ob0, ob1)
        gsems = (g0, g1)
        wsems = (w0, w1)
        iota = lax.iota(jnp.int32, LANES)
        rbs = [iota + ri * LANES for ri in range(rblk // LANES)]

        pltpu.sync_copy(xt_hbm.at[:, pl.ds(r0, rblk)], idxv)

        def start_gather(t, b):
            pltpu.async_copy(wide_hbm.at[idxv.at[t]], ins[b], gsems[b])

        for b in range(2):
            start_gather(b, b)

        def t_body(r, carry):
            for b in range(2):
                t = r * 2 + b
                pltpu.make_async_copy(
                    wide_hbm.at[idxv.at[0]], ins[b], gsems[b]).wait()

                @pl.when(t >= 2)
                def _():
                    pltpu.make_async_copy(
                        obs[b], out_hbm.at[0, :, pl.ds(0, rblk)],
                        wsems[b]).wait()

                def diag_body(c, cr):
                    pc = (iota + c) & (LANES - 1)
                    for ri in range(rblk // LANES):
                        rb = rbs[ri]
                        for dj in range(D_MODEL // LANES):
                            dcol = pc + dj * LANES
                            v = plsc.load_gather(ins[b], [rb, dcol]) * SCALE
                            plsc.store_scatter(obs[b], [dcol, rb], v)
                    return cr

                lax.fori_loop(0, LANES, diag_body, 0)

                @pl.when(t + 2 < n_pos)
                def _():
                    start_gather(t + 2, b)

                pltpu.async_copy(
                    obs[b], out_hbm.at[t, :, pl.ds(r0, rblk)], wsems[b])
            return carry

        lax.fori_loop(0, n_pos // 2, t_body, 0)

        for bb in range(2):
            pltpu.make_async_copy(
                obs[bb], out_hbm.at[0, :, pl.ds(0, rblk)], wsems[bb]).wait()

    return k2(x_t, wide)


def kernel(x, table):
    x_t = x.astype(jnp.int32).T
    table_t = table.T
    wide = _widen(table_t)
    out_t = _gather(x_t, wide)
    return jnp.transpose(out_t, (2, 0, 1))
